# baseline (speedup 1.0000x reference)
import functools

import jax
import jax.numpy as jnp
from jax import lax
from jax.experimental import pallas as pl
from jax.experimental.pallas import tpu as pltpu
from jax.experimental.pallas import tpu_sc as plsc

_NC = 1
_NS = 16
_NW = _NC * _NS
_L = 16


def _make_sc_call(n_elems):
    per_w = n_elems // _NW
    n_vec = per_w // _L

    def _body(enc_hbm, dates_hbm, src_hbm, out_hbm, table_v, dates_v, src_v):
        wid = lax.axis_index("s") * _NC + lax.axis_index("c")
        pltpu.sync_copy(enc_hbm, table_v)
        pltpu.sync_copy(dates_hbm.at[pl.ds(wid * (2 * per_w), 2 * per_w)], dates_v)
        pltpu.sync_copy(src_hbm.at[pl.ds(wid * per_w, per_w)], src_v)
        lanes2 = lax.iota(jnp.int32, 16) * 2
        for i in range(n_vec):
            m = plsc.load_gather(dates_v, [lanes2 + (2 * _L * i)])
            d = plsc.load_gather(dates_v, [lanes2 + (2 * _L * i + 1)])
            enc = plsc.load_gather(table_v, [m * 32 + d])
            src_v[pl.ds(i * _L, _L)] = src_v[pl.ds(i * _L, _L)] + enc
        pltpu.sync_copy(src_v, out_hbm.at[pl.ds(wid * per_w, per_w)])

    return pl.kernel(
        _body,
        out_type=jax.ShapeDtypeStruct((n_elems,), jnp.float32),
        mesh=plsc.VectorSubcoreMesh(
            core_axis_name="c", subcore_axis_name="s", num_cores=_NC),
        scratch_types=[
            pltpu.VMEM((12 * 32,), jnp.float32),
            pltpu.VMEM((2 * per_w,), jnp.int32),
            pltpu.VMEM((per_w,), jnp.float32),
        ],
        compiler_params=pltpu.CompilerParams(
            needs_layout_passes=False, skip_device_barrier=True),
    )


def kernel(src, dates, encoding):
    b, s = src.shape
    n = b * s
    enc_pad = jnp.pad(encoding.astype(jnp.float32), ((0, 0), (0, 1)))
    out = _make_sc_call(n)(enc_pad.reshape(-1), dates.reshape(-1), src.reshape(-1))
    return out.reshape(b, s)


# speedup vs baseline: 1.9676x; 1.9676x over previous
import jax
import jax.numpy as jnp
from jax import lax
from jax.experimental import pallas as pl
from jax.experimental.pallas import tpu as pltpu
from jax.experimental.pallas import tpu_sc as plsc

_NC = 2
_NS = 16
_NW = _NC * _NS
_L = 16


def _make_sc_call(nt):
    tpw = nt // _NW

    def _body(enc_hbm, dates_hbm, src_hbm, out_hbm, table_v, dates_v, src_v):
        wid = lax.axis_index("s") * _NC + lax.axis_index("c")
        pltpu.sync_copy(enc_hbm, table_v)
        pltpu.sync_copy(dates_hbm.at[:, pl.ds(wid * tpw, tpw)], dates_v)
        pltpu.sync_copy(src_hbm.at[pl.ds(wid * tpw, tpw)], src_v)
        for ci in range(tpw):
            for r in range(4):
                for g in range(128 // _L):
                    m = dates_v[r, ci, 0, pl.ds(g * _L, _L)]
                    d = dates_v[r, ci, 1, pl.ds(g * _L, _L)]
                    e = plsc.load_gather(table_v, [m, d])
                    src_v[ci, r, pl.ds(g * _L, _L)] = (
                        src_v[ci, r, pl.ds(g * _L, _L)] + e)
        pltpu.sync_copy(src_v, out_hbm.at[pl.ds(wid * tpw, tpw)])

    return pl.kernel(
        _body,
        out_type=jax.ShapeDtypeStruct((nt, 4, 128), jnp.float32),
        mesh=plsc.VectorSubcoreMesh(
            core_axis_name="c", subcore_axis_name="s", num_cores=_NC),
        scratch_types=[
            pltpu.VMEM((12, 32), jnp.float32),
            pltpu.VMEM((4, tpw, 2, 128), jnp.int32),
            pltpu.VMEM((tpw, 4, 128), jnp.float32),
        ],
        compiler_params=pltpu.CompilerParams(needs_layout_passes=False),
    )


def kernel(src, dates, encoding):
    b, s = src.shape
    nt = s // 128
    src_p = src.reshape(b, nt, 128).transpose(1, 0, 2)
    dates_p = dates.reshape(b, nt, 128, 2).transpose(0, 1, 3, 2)
    enc_pad = jnp.pad(encoding.astype(jnp.float32), ((0, 0), (0, 1)))
    out_p = _make_sc_call(nt)(enc_pad, dates_p, src_p)
    return out_p.transpose(1, 0, 2).reshape(b, s)
